# baseline (speedup 1.0000x reference)
import jax
import jax.numpy as jnp
from jax import lax
from jax.experimental import pallas as pl
from jax.experimental.pallas import tpu as pltpu
from jax.experimental.pallas import tpu_sc as plsc

N_NODES = 100000
NODE_DIM = 128
N_EDGES = 1600000

NC, NS, L = 2, 16, 16
NW = NC * NS

ROW = 128
CH = 6272
NCH = 16
EDGES_PER_TILE = CH * NCH
E_TOTAL = NS * EDGES_PER_TILE
E_PAD = E_TOTAL - N_EDGES
UNROLL = 4

HIST = 100096
MSLICE = HIST // NS

N_FULL_CHUNKS = N_NODES // ROW
REM = N_NODES - N_FULL_CHUNKS * ROW
REM_BASE = N_FULL_CHUNKS * ROW
K_ITERS = (N_FULL_CHUNKS + NW - 1) // NW
PAIRS = (K_ITERS + 2) // 2


def _body(x_hbm, dst_hbm, emb_hbm, out_hbm, phist_hbm,
          hist, sem_s0, sem_s1, sem_d0, sem_d1, sem_g0, sem_g1,
          sem_x0, sem_x1, sem_w0, sem_w1):
    s = lax.axis_index("s")
    c = lax.axis_index("c")
    w = s * NC + c
    sem_s = (sem_s0, sem_s1)

    def _phase_a(hist_priv, st0, st1):
        st = (st0, st1)

        with jax.named_scope("ph_a_zero"):
            def _z(i, _):
                hist_priv[pl.ds(i * L, L)] = jnp.zeros((L,), jnp.int32)
                return 0
            lax.fori_loop(0, HIST // L, _z, 0)

        ebase = s * EDGES_PER_TILE
        with jax.named_scope("ph_a_build"):
            pltpu.sync_copy(dst_hbm.at[pl.ds(ebase, CH)], st0)
            for m in range(NCH):
                cur = st[m % 2]
                if m + 1 < NCH:
                    cp = pltpu.async_copy(
                        dst_hbm.at[pl.ds(ebase + (m + 1) * CH, CH)],
                        st[(m + 1) % 2], sem_s[(m + 1) % 2])

                def _vec(v, _):
                    for u in range(UNROLL):
                        iv = cur[pl.ds((v * UNROLL + u) * L, L)]
                        cnt, last = plsc.scan_count(iv)
                        plsc.addupdate_scatter(hist_priv, [iv], cnt, mask=last)
                    return 0
                lax.fori_loop(0, CH // L // UNROLL, _vec, 0)
                if m + 1 < NCH:
                    cp.wait()

        with jax.named_scope("ph_a_merge"):
            pltpu.sync_copy(hist_priv,
                            phist_hbm.at[pl.ds((c * NS + s) * HIST, HIST)])
            plsc.subcore_barrier()

            my_off = s * MSLICE

            def _peer_src(t, b):
                tt = lax.rem(s + 1 + t, NS)
                return pltpu.make_async_copy(
                    phist_hbm.at[pl.ds((c * NS + tt) * HIST + my_off, MSLICE)],
                    st[b].at[pl.ds(0, MSLICE)], sem_s[b])

            _peer_src(0, 0).start()
            for t in range(NS - 1):
                if t + 1 < NS - 1:
                    _peer_src(t + 1, (t + 1) % 2).start()
                _peer_src(t, t % 2).wait()

                def _acc(i, _):
                    plsc.addupdate(hist_priv.at[pl.ds(my_off + i * L, L)],
                                   st[t % 2][pl.ds(i * L, L)])
                    return 0
                lax.fori_loop(0, MSLICE // L, _acc, 0)

            pltpu.sync_copy(hist_priv.at[pl.ds(my_off, MSLICE)],
                            hist.at[pl.ds(my_off, MSLICE)])

    pl.run_scoped(_phase_a,
                  pltpu.VMEM((HIST,), jnp.int32),
                  pltpu.VMEM((CH,), jnp.int32),
                  pltpu.VMEM((CH,), jnp.int32))
    plsc.subcore_barrier()

    lane = lax.iota(jnp.int32, L)
    fix_vec = (1 - jnp.minimum(lane, 1)) * E_PAD

    def _phase_b(deg0, deg1, buf_e0, buf_e1, buf_x0, buf_x1):
        deg = (deg0, deg1)
        buf_e = (buf_e0, buf_e1)
        buf_x = (buf_x0, buf_x1)
        sem_d = (sem_d0, sem_d1)
        sem_g = (sem_g0, sem_g1)
        sem_x = (sem_x0, sem_x1)
        sem_w = (sem_w0, sem_w1)

        def _valid(t):
            return w + NW * t < N_FULL_CHUNKS

        def _stage_deg(b, t):
            pltpu.async_copy(hist.at[pl.ds((w + NW * t) * ROW, ROW)],
                             deg[b], sem_d[b])

        def _wait_deg(b, t):
            pltpu.make_async_copy(hist.at[pl.ds((w + NW * t) * ROW, ROW)],
                                  deg[b], sem_d[b]).wait()

        def _clamp(b, t):
            @pl.when(w + NW * t == 0)
            def _():
                deg[b][pl.ds(0, L)] = deg[b][pl.ds(0, L)] - fix_vec
            for i in range(ROW // L):
                deg[b][pl.ds(i * L, L)] = jnp.minimum(
                    deg[b][pl.ds(i * L, L)], N_NODES - 1)

        def _start_gx(b, t):
            base = (w + NW * t) * ROW
            pltpu.async_copy(emb_hbm.at[deg[b]], buf_e[b], sem_g[b])
            pltpu.async_copy(x_hbm.at[pl.ds(base, ROW)], buf_x[b], sem_x[b])

        def _wait_gx(b, t):
            base = (w + NW * t) * ROW
            pltpu.make_async_copy(emb_hbm.at[deg[b]], buf_e[b],
                                  sem_g[b]).wait()
            pltpu.make_async_copy(x_hbm.at[pl.ds(base, ROW)], buf_x[b],
                                  sem_x[b]).wait()

        def _add(b):
            def _rows(r, _):
                for u in range(2):
                    for i in range(NODE_DIM // L):
                        plsc.addupdate(
                            buf_e[b].at[2 * r + u, pl.ds(i * L, L)],
                            buf_x[b][2 * r + u, pl.ds(i * L, L)])
                return 0
            lax.fori_loop(0, ROW // 2, _rows, 0)

        def _start_write(b, t):
            base = (w + NW * t) * ROW
            pltpu.async_copy(buf_e[b], out_hbm.at[pl.ds(base, ROW)], sem_w[b])

        def _wait_write(b, t):
            base = (w + NW * t) * ROW
            pltpu.make_async_copy(buf_e[b], out_hbm.at[pl.ds(base, ROW)],
                                  sem_w[b]).wait()

        def _process(b, t):
            _wait_gx(b, t)

            @pl.when(_valid(t + 2))
            def _():
                _stage_deg(b, t + 2)

            @pl.when(t >= 1)
            def _():
                _wait_write(1 - b, t - 1)

            @pl.when(_valid(t + 1))
            def _():
                _wait_deg(1 - b, t + 1)
                _clamp(1 - b, t + 1)
                _start_gx(1 - b, t + 1)

            _add(b)
            _start_write(b, t)

        _stage_deg(0, 0)

        @pl.when(_valid(1))
        def _():
            _stage_deg(1, 1)
        _wait_deg(0, 0)
        _clamp(0, 0)
        _start_gx(0, 0)

        def _pair(tp, _):
            t0 = 2 * tp

            @pl.when(_valid(t0))
            def _():
                _process(0, t0)

            @pl.when(_valid(t0 + 1))
            def _():
                _process(1, t0 + 1)
            return 0
        lax.fori_loop(0, PAIRS, _pair, 0)

        @pl.when(w <= (N_FULL_CHUNKS - 1) % NW)
        def _():
            _wait_write(0, K_ITERS - 1)

        @pl.when(w > (N_FULL_CHUNKS - 1) % NW)
        def _():
            _wait_write(1, K_ITERS - 2)

        @pl.when(w == 13)
        def _():
            pltpu.sync_copy(hist.at[pl.ds(REM_BASE, REM)],
                            deg0.at[pl.ds(0, REM)])
            for i in range(REM // L):
                v = deg0[pl.ds(i * L, L)]
                deg0[pl.ds(i * L, L)] = jnp.minimum(v, N_NODES - 1)
            cp_g = pltpu.async_copy(emb_hbm.at[deg0.at[pl.ds(0, REM)]],
                                    buf_e0.at[pl.ds(0, REM)], sem_g0)
            cp_x = pltpu.async_copy(x_hbm.at[pl.ds(REM_BASE, REM)],
                                    buf_x0.at[pl.ds(0, REM)], sem_x0)
            cp_g.wait()
            cp_x.wait()

            def _radd(r, _):
                for i in range(NODE_DIM // L):
                    plsc.addupdate(buf_e0.at[r, pl.ds(i * L, L)],
                                   buf_x0[r, pl.ds(i * L, L)])
                return 0
            lax.fori_loop(0, REM, _radd, 0)
            pltpu.sync_copy(buf_e0.at[pl.ds(0, REM)],
                            out_hbm.at[pl.ds(REM_BASE, REM)])

    with jax.named_scope("ph_b"):
        pl.run_scoped(_phase_b,
                  pltpu.VMEM((ROW,), jnp.int32),
                  pltpu.VMEM((ROW,), jnp.int32),
                  pltpu.VMEM((ROW, NODE_DIM), jnp.float32),
                  pltpu.VMEM((ROW, NODE_DIM), jnp.float32),
                  pltpu.VMEM((ROW, NODE_DIM), jnp.float32),
                  pltpu.VMEM((ROW, NODE_DIM), jnp.float32))


_sc_call = pl.kernel(
    _body,
    out_type=(jax.ShapeDtypeStruct((N_NODES, NODE_DIM), jnp.float32),
              jax.ShapeDtypeStruct((NC * NS * HIST,), jnp.int32)),
    mesh=plsc.VectorSubcoreMesh(core_axis_name="c", subcore_axis_name="s",
                                num_cores=NC, num_subcores=NS),
    scratch_types=[
        pltpu.VMEM_SHARED((HIST,), jnp.int32),
    ] + [pltpu.SemaphoreType.DMA] * 10,
    compiler_params=pltpu.CompilerParams(needs_layout_passes=False),
)


@jax.jit
def kernel(x, edge_index, degree_embedding):
    dst = edge_index[1].astype(jnp.int32)
    dst = jnp.concatenate([dst, jnp.zeros((E_PAD,), jnp.int32)])
    out, _ = _sc_call(x, dst, degree_embedding)
    return out


# speedup vs baseline: 2.5842x; 2.5842x over previous
import jax
import jax.numpy as jnp
from jax import lax
from jax.experimental import pallas as pl
from jax.experimental.pallas import tpu as pltpu
from jax.experimental.pallas import tpu_sc as plsc

N_NODES = 100000
NODE_DIM = 128
N_EDGES = 1600000

NC, NS, L = 2, 16, 16
NW = NC * NS

CH = 2048
NCH = 49
EDGES_PER_TILE = CH * NCH
E_TOTAL = NS * EDGES_PER_TILE
E_PAD = E_TOTAL - N_EDGES
UNROLL = 4

HIST = 100096
HHALF = HIST // 2
MSLICE = HIST // NS

CACHE_N = 4096
CROWS = CACHE_N // NS

ROWB = 96
N_FULL_CHUNKS = N_NODES // ROWB
REM = N_NODES - N_FULL_CHUNKS * ROWB
REM_BASE = N_FULL_CHUNKS * ROWB
K_ITERS = (N_FULL_CHUNKS + NW - 1) // NW
PAIRS = (K_ITERS + 2) // 2


def _body(x_hbm, dst_hbm, emb_hbm, out_hbm, phist_hbm,
          hist, cache, sem_s0, sem_s1, sem_d0, sem_d1, sem_g0, sem_g1,
          sem_x0, sem_x1, sem_w0, sem_w1):
    s = lax.axis_index("s")
    c = lax.axis_index("c")
    w = s * NC + c
    sem_s = (sem_s0, sem_s1)

    pltpu.sync_copy(emb_hbm.at[pl.ds(s * CROWS, CROWS)],
                    cache.at[pl.ds(s * CROWS, CROWS)])

    def _phase_a(hist_priv, st0, st1):
        st = (st0, st1)
        ebase = s * EDGES_PER_TILE

        for p in range(2):
            lo = p * HHALF

            with jax.named_scope("ph_a_zero"):
                def _z(i, _):
                    for u in range(8):
                        hist_priv[pl.ds((i * 8 + u) * L, L)] = (
                            jnp.zeros((L,), jnp.int32))
                    return 0
                lax.fori_loop(0, HHALF // L // 8, _z, 0)

            with jax.named_scope("ph_a_build"):
                pltpu.sync_copy(dst_hbm.at[pl.ds(ebase, CH)], st0)
                for m in range(NCH):
                    cur = st[m % 2]
                    if m + 1 < NCH:
                        cp = pltpu.async_copy(
                            dst_hbm.at[pl.ds(ebase + (m + 1) * CH, CH)],
                            st[(m + 1) % 2], sem_s[(m + 1) % 2])

                    def _vec(v, _):
                        for u in range(UNROLL):
                            iv = cur[pl.ds((v * UNROLL + u) * L, L)]
                            local = iv - lo
                            inr = (local >= 0) & (local < HHALF)
                            lc = jnp.minimum(jnp.maximum(local, 0), HHALF - 1)
                            cnt, last = plsc.scan_count(iv)
                            plsc.addupdate_scatter(hist_priv, [lc], cnt,
                                                   mask=last & inr)
                        return 0
                    lax.fori_loop(0, CH // L // UNROLL, _vec, 0)
                    if m + 1 < NCH:
                        cp.wait()

                pltpu.sync_copy(
                    hist_priv,
                    phist_hbm.at[pl.ds((c * NS + s) * HIST + lo, HHALF)])

        plsc.subcore_barrier()

        acc = hist_priv.at[pl.ds(0, MSLICE)]
        mt = (hist_priv.at[pl.ds(MSLICE, MSLICE)],
              hist_priv.at[pl.ds(2 * MSLICE, MSLICE)])
        my_off = s * MSLICE

        with jax.named_scope("ph_a_merge"):
            def _src(t):
                tt = lax.rem(s + t, NS)
                return phist_hbm.at[
                    pl.ds((c * NS + tt) * HIST + my_off, MSLICE)]

            pltpu.sync_copy(_src(0), acc)
            pltpu.async_copy(_src(1), mt[1], sem_s[1])
            for t in range(1, NS):
                if t + 1 < NS:
                    pltpu.async_copy(_src(t + 1), mt[(t + 1) % 2],
                                     sem_s[(t + 1) % 2])
                pltpu.make_async_copy(_src(t), mt[t % 2], sem_s[t % 2]).wait()

                def _acc(i, _):
                    plsc.addupdate(acc.at[pl.ds(i * L, L)],
                                   mt[t % 2][pl.ds(i * L, L)])
                    return 0
                lax.fori_loop(0, MSLICE // L, _acc, 0)

            pltpu.sync_copy(acc, hist.at[pl.ds(my_off, MSLICE)])

    pl.run_scoped(_phase_a,
                  pltpu.VMEM((HHALF,), jnp.int32),
                  pltpu.VMEM((CH,), jnp.int32),
                  pltpu.VMEM((CH,), jnp.int32))
    plsc.subcore_barrier()

    lane = lax.iota(jnp.int32, L)
    fix_vec = (1 - jnp.minimum(lane, 1)) * E_PAD

    def _phase_b(deg0, deg1, buf_e0, buf_e1, buf_x0, buf_x1):
        deg = (deg0, deg1)
        buf_e = (buf_e0, buf_e1)
        buf_x = (buf_x0, buf_x1)
        sem_d = (sem_d0, sem_d1)
        sem_g = (sem_g0, sem_g1)
        sem_x = (sem_x0, sem_x1)
        sem_w = (sem_w0, sem_w1)

        def _valid(t):
            return w + NW * t < N_FULL_CHUNKS

        def _stage_deg(b, t):
            pltpu.async_copy(hist.at[pl.ds((w + NW * t) * ROWB, ROWB)],
                             deg[b], sem_d[b])

        def _wait_deg(b, t):
            pltpu.make_async_copy(hist.at[pl.ds((w + NW * t) * ROWB, ROWB)],
                                  deg[b], sem_d[b]).wait()

        def _clamp(b, t):
            @pl.when(w + NW * t == 0)
            def _():
                deg[b][pl.ds(0, L)] = deg[b][pl.ds(0, L)] - fix_vec
            mx = jnp.zeros((L,), jnp.int32)
            for i in range(ROWB // L):
                v = jnp.minimum(deg[b][pl.ds(i * L, L)], N_NODES - 1)
                deg[b][pl.ds(i * L, L)] = v
                mx = jnp.maximum(mx, v)
            return jnp.max(mx)

        def _start_gx(b, t, dmax):
            base = (w + NW * t) * ROWB

            @pl.when(dmax < CACHE_N)
            def _():
                pltpu.async_copy(cache.at[deg[b]], buf_e[b], sem_g[b])

            @pl.when(dmax >= CACHE_N)
            def _():
                pltpu.async_copy(emb_hbm.at[deg[b]], buf_e[b], sem_g[b])
            pltpu.async_copy(x_hbm.at[pl.ds(base, ROWB)], buf_x[b], sem_x[b])

        def _wait_gx(b, t):
            base = (w + NW * t) * ROWB
            pltpu.make_async_copy(emb_hbm.at[deg[b]], buf_e[b],
                                  sem_g[b]).wait()
            pltpu.make_async_copy(x_hbm.at[pl.ds(base, ROWB)], buf_x[b],
                                  sem_x[b]).wait()

        def _add(b):
            def _rows(r, _):
                for u in range(2):
                    for i in range(NODE_DIM // L):
                        plsc.addupdate(
                            buf_e[b].at[2 * r + u, pl.ds(i * L, L)],
                            buf_x[b][2 * r + u, pl.ds(i * L, L)])
                return 0
            lax.fori_loop(0, ROWB // 2, _rows, 0)

        def _start_write(b, t):
            base = (w + NW * t) * ROWB
            pltpu.async_copy(buf_e[b], out_hbm.at[pl.ds(base, ROWB)],
                             sem_w[b])

        def _wait_write(b, t):
            base = (w + NW * t) * ROWB
            pltpu.make_async_copy(buf_e[b], out_hbm.at[pl.ds(base, ROWB)],
                                  sem_w[b]).wait()

        def _process(b, t):
            _wait_gx(b, t)

            @pl.when(_valid(t + 2))
            def _():
                _stage_deg(b, t + 2)

            @pl.when(t >= 1)
            def _():
                _wait_write(1 - b, t - 1)

            @pl.when(_valid(t + 1))
            def _():
                _wait_deg(1 - b, t + 1)
                dmax = _clamp(1 - b, t + 1)
                _start_gx(1 - b, t + 1, dmax)

            _add(b)
            _start_write(b, t)

        _stage_deg(0, 0)

        @pl.when(_valid(1))
        def _():
            _stage_deg(1, 1)
        _wait_deg(0, 0)
        dmax0 = _clamp(0, 0)
        _start_gx(0, 0, dmax0)

        def _pair(tp, _):
            t0 = 2 * tp

            @pl.when(_valid(t0))
            def _():
                _process(0, t0)

            @pl.when(_valid(t0 + 1))
            def _():
                _process(1, t0 + 1)
            return 0
        lax.fori_loop(0, PAIRS, _pair, 0)

        @pl.when(w <= (N_FULL_CHUNKS - 1) % NW)
        def _():
            _wait_write(0, K_ITERS - 1)

        @pl.when(w > (N_FULL_CHUNKS - 1) % NW)
        def _():
            _wait_write(1, K_ITERS - 2)

        @pl.when(w == 13)
        def _():
            pltpu.sync_copy(hist.at[pl.ds(REM_BASE, REM)],
                            deg0.at[pl.ds(0, REM)])
            mx = jnp.zeros((L,), jnp.int32)
            for i in range(REM // L):
                v = jnp.minimum(deg0[pl.ds(i * L, L)], N_NODES - 1)
                deg0[pl.ds(i * L, L)] = v
                mx = jnp.maximum(mx, v)
            rmax = jnp.max(mx)

            @pl.when(rmax < CACHE_N)
            def _():
                pltpu.async_copy(cache.at[deg0.at[pl.ds(0, REM)]],
                                 buf_e0.at[pl.ds(0, REM)], sem_g0)

            @pl.when(rmax >= CACHE_N)
            def _():
                pltpu.async_copy(emb_hbm.at[deg0.at[pl.ds(0, REM)]],
                                 buf_e0.at[pl.ds(0, REM)], sem_g0)
            cp_x = pltpu.async_copy(x_hbm.at[pl.ds(REM_BASE, REM)],
                                    buf_x0.at[pl.ds(0, REM)], sem_x0)
            pltpu.make_async_copy(emb_hbm.at[deg0.at[pl.ds(0, REM)]],
                                  buf_e0.at[pl.ds(0, REM)], sem_g0).wait()
            cp_x.wait()

            def _radd(r, _):
                for i in range(NODE_DIM // L):
                    plsc.addupdate(buf_e0.at[r, pl.ds(i * L, L)],
                                   buf_x0[r, pl.ds(i * L, L)])
                return 0
            lax.fori_loop(0, REM, _radd, 0)
            pltpu.sync_copy(buf_e0.at[pl.ds(0, REM)],
                            out_hbm.at[pl.ds(REM_BASE, REM)])

    with jax.named_scope("ph_b"):
        pl.run_scoped(_phase_b,
                      pltpu.VMEM((ROWB,), jnp.int32),
                      pltpu.VMEM((ROWB,), jnp.int32),
                      pltpu.VMEM((ROWB, NODE_DIM), jnp.float32),
                      pltpu.VMEM((ROWB, NODE_DIM), jnp.float32),
                      pltpu.VMEM((ROWB, NODE_DIM), jnp.float32),
                      pltpu.VMEM((ROWB, NODE_DIM), jnp.float32))


_sc_call = pl.kernel(
    _body,
    out_type=(jax.ShapeDtypeStruct((N_NODES, NODE_DIM), jnp.float32),
              jax.ShapeDtypeStruct((NC * NS * HIST,), jnp.int32)),
    mesh=plsc.VectorSubcoreMesh(core_axis_name="c", subcore_axis_name="s",
                                num_cores=NC, num_subcores=NS),
    scratch_types=[
        pltpu.VMEM_SHARED((HIST,), jnp.int32),
        pltpu.VMEM_SHARED((CACHE_N, NODE_DIM), jnp.float32),
    ] + [pltpu.SemaphoreType.DMA] * 10,
    compiler_params=pltpu.CompilerParams(needs_layout_passes=False),
)


@jax.jit
def kernel(x, edge_index, degree_embedding):
    dst = edge_index[1].astype(jnp.int32)
    dst = jnp.concatenate([dst, jnp.zeros((E_PAD,), jnp.int32)])
    out, _ = _sc_call(x, dst, degree_embedding)
    return out


# speedup vs baseline: 3.2906x; 1.2734x over previous
import jax
import jax.numpy as jnp
from jax import lax
from jax.experimental import pallas as pl
from jax.experimental.pallas import tpu as pltpu
from jax.experimental.pallas import tpu_sc as plsc

N_NODES = 100000
NODE_DIM = 128
N_EDGES = 1600000

NC, NS, L = 2, 16, 16
NW = NC * NS

CH = 2048
EDGES_PER_TILE = N_EDGES // NS
NCH_FULL = EDGES_PER_TILE // CH
TAIL = EDGES_PER_TILE - NCH_FULL * CH
UNROLL = 4

HIST = 100096
HHALF = HIST // 2
MSLICE = HIST // NS

CACHE_N = 4096
CROWS = CACHE_N // NS

ROWB = 96
N_FULL_CHUNKS = N_NODES // ROWB
REM = N_NODES - N_FULL_CHUNKS * ROWB
REM_BASE = N_FULL_CHUNKS * ROWB
K_ITERS = (N_FULL_CHUNKS + NW - 1) // NW
PAIRS = (K_ITERS + 2) // 2


def _body(x_hbm, dst_hbm, emb_hbm, out_hbm, phist_hbm,
          hist, cache, sem_s0, sem_s1, sem_d0, sem_d1, sem_g0, sem_g1,
          sem_x0, sem_x1, sem_w0, sem_w1):
    s = lax.axis_index("s")
    c = lax.axis_index("c")
    w = s * NC + c
    sem_s = (sem_s0, sem_s1)

    pltpu.sync_copy(emb_hbm.at[pl.ds(s * CROWS, CROWS)],
                    cache.at[pl.ds(s * CROWS, CROWS)])

    def _phase_a(hist_priv, st0, st1):
        st = (st0, st1)
        ebase = N_EDGES + s * EDGES_PER_TILE

        def _stage(k, b, sem):
            return pltpu.async_copy(dst_hbm.at[pl.ds(ebase + k * CH, CH)],
                                    st[b], sem)

        for p in range(2):
            lo = p * HHALF
            ulo = jnp.uint32(lo)
            uhh = jnp.uint32(HHALF)

            with jax.named_scope("ph_a_zero"):
                def _z(i, _):
                    for u in range(8):
                        hist_priv[pl.ds((i * 8 + u) * L, L)] = (
                            jnp.zeros((L,), jnp.int32))
                    return 0
                lax.fori_loop(0, HHALF // L // 8, _z, 0)

            def _scan(cur, nvec_div):
                def _vec(v, _):
                    for u in range(UNROLL):
                        iv = cur[pl.ds((v * UNROLL + u) * L, L)]
                        local = iv - lo
                        inr = (plsc.bitcast(iv, jnp.uint32) - ulo) < uhh
                        cnt, last = plsc.scan_count(iv)
                        plsc.addupdate_scatter(hist_priv, [local], cnt,
                                               mask=last & inr)
                    return 0
                lax.fori_loop(0, nvec_div, _vec, 0)

            with jax.named_scope("ph_a_build"):
                _stage(0, 0, sem_s0).wait()

                def _chunk_pair(cp, _):
                    k0 = 2 * cp
                    _stage(k0 + 1, 1, sem_s1)
                    _scan(st0, CH // L // UNROLL)
                    pltpu.make_async_copy(
                        dst_hbm.at[pl.ds(ebase, CH)], st1, sem_s1).wait()

                    @pl.when(k0 + 2 < NCH_FULL)
                    def _():
                        _stage(k0 + 2, 0, sem_s0)
                    _scan(st1, CH // L // UNROLL)

                    @pl.when(k0 + 2 < NCH_FULL)
                    def _():
                        pltpu.make_async_copy(
                            dst_hbm.at[pl.ds(ebase, CH)], st0, sem_s0).wait()
                    return 0
                lax.fori_loop(0, NCH_FULL // 2, _chunk_pair, 0)

                pltpu.sync_copy(
                    dst_hbm.at[pl.ds(ebase + NCH_FULL * CH, TAIL)],
                    st0.at[pl.ds(0, TAIL)])
                _scan(st0, TAIL // L // UNROLL)
                rem_v = (TAIL // L) % UNROLL
                for u in range(rem_v):
                    iv = st0[pl.ds((TAIL // L - rem_v + u) * L, L)]
                    local = iv - lo
                    inr = (plsc.bitcast(iv, jnp.uint32) - ulo) < uhh
                    cnt, last = plsc.scan_count(iv)
                    plsc.addupdate_scatter(hist_priv, [local], cnt,
                                           mask=last & inr)

                pltpu.sync_copy(
                    hist_priv,
                    phist_hbm.at[pl.ds((c * NS + s) * HIST + lo, HHALF)])

        plsc.subcore_barrier()

        acc = hist_priv.at[pl.ds(0, MSLICE)]
        mt = (hist_priv.at[pl.ds(MSLICE, MSLICE)],
              hist_priv.at[pl.ds(2 * MSLICE, MSLICE)])
        my_off = s * MSLICE

        with jax.named_scope("ph_a_merge"):
            def _src(t):
                tt = lax.rem(s + t, NS)
                return phist_hbm.at[
                    pl.ds((c * NS + tt) * HIST + my_off, MSLICE)]

            pltpu.sync_copy(_src(0), acc)
            pltpu.async_copy(_src(1), mt[1], sem_s[1])
            for t in range(1, NS):
                if t + 1 < NS:
                    pltpu.async_copy(_src(t + 1), mt[(t + 1) % 2],
                                     sem_s[(t + 1) % 2])
                pltpu.make_async_copy(_src(t), mt[t % 2], sem_s[t % 2]).wait()

                def _acc(i, _):
                    plsc.addupdate(acc.at[pl.ds(i * L, L)],
                                   mt[t % 2][pl.ds(i * L, L)])
                    return 0
                lax.fori_loop(0, MSLICE // L, _acc, 0)

            pltpu.sync_copy(acc, hist.at[pl.ds(my_off, MSLICE)])

    pl.run_scoped(_phase_a,
                  pltpu.VMEM((HHALF,), jnp.int32),
                  pltpu.VMEM((CH,), jnp.int32),
                  pltpu.VMEM((CH,), jnp.int32))
    plsc.subcore_barrier()

    def _phase_b(deg0, deg1, buf_e0, buf_e1, buf_x0, buf_x1):
        deg = (deg0, deg1)
        buf_e = (buf_e0, buf_e1)
        buf_x = (buf_x0, buf_x1)
        sem_d = (sem_d0, sem_d1)
        sem_g = (sem_g0, sem_g1)
        sem_x = (sem_x0, sem_x1)
        sem_w = (sem_w0, sem_w1)

        def _valid(t):
            return w + NW * t < N_FULL_CHUNKS

        def _stage_deg(b, t):
            pltpu.async_copy(hist.at[pl.ds((w + NW * t) * ROWB, ROWB)],
                             deg[b], sem_d[b])

        def _wait_deg(b, t):
            pltpu.make_async_copy(hist.at[pl.ds((w + NW * t) * ROWB, ROWB)],
                                  deg[b], sem_d[b]).wait()

        def _clamp(b, t):
            mx = jnp.zeros((L,), jnp.int32)
            for i in range(ROWB // L):
                v = jnp.minimum(deg[b][pl.ds(i * L, L)], N_NODES - 1)
                deg[b][pl.ds(i * L, L)] = v
                mx = jnp.maximum(mx, v)
            return jnp.max(mx)

        def _start_gx(b, t, dmax):
            base = (w + NW * t) * ROWB

            @pl.when(dmax < CACHE_N)
            def _():
                pltpu.async_copy(cache.at[deg[b]], buf_e[b], sem_g[b])

            @pl.when(dmax >= CACHE_N)
            def _():
                pltpu.async_copy(emb_hbm.at[deg[b]], buf_e[b], sem_g[b])
            pltpu.async_copy(x_hbm.at[pl.ds(base, ROWB)], buf_x[b], sem_x[b])

        def _wait_gx(b, t):
            base = (w + NW * t) * ROWB
            pltpu.make_async_copy(emb_hbm.at[deg[b]], buf_e[b],
                                  sem_g[b]).wait()
            pltpu.make_async_copy(x_hbm.at[pl.ds(base, ROWB)], buf_x[b],
                                  sem_x[b]).wait()

        def _add(b):
            def _rows(r, _):
                for u in range(2):
                    for i in range(NODE_DIM // L):
                        plsc.addupdate(
                            buf_e[b].at[2 * r + u, pl.ds(i * L, L)],
                            buf_x[b][2 * r + u, pl.ds(i * L, L)])
                return 0
            lax.fori_loop(0, ROWB // 2, _rows, 0)

        def _start_write(b, t):
            base = (w + NW * t) * ROWB
            pltpu.async_copy(buf_e[b], out_hbm.at[pl.ds(base, ROWB)],
                             sem_w[b])

        def _wait_write(b, t):
            base = (w + NW * t) * ROWB
            pltpu.make_async_copy(buf_e[b], out_hbm.at[pl.ds(base, ROWB)],
                                  sem_w[b]).wait()

        def _process(b, t):
            _wait_gx(b, t)

            @pl.when(_valid(t + 2))
            def _():
                _stage_deg(b, t + 2)

            @pl.when(t >= 1)
            def _():
                _wait_write(1 - b, t - 1)

            @pl.when(_valid(t + 1))
            def _():
                _wait_deg(1 - b, t + 1)
                dmax = _clamp(1 - b, t + 1)
                _start_gx(1 - b, t + 1, dmax)

            _add(b)
            _start_write(b, t)

        _stage_deg(0, 0)

        @pl.when(_valid(1))
        def _():
            _stage_deg(1, 1)
        _wait_deg(0, 0)
        dmax0 = _clamp(0, 0)
        _start_gx(0, 0, dmax0)

        def _pair(tp, _):
            t0 = 2 * tp

            @pl.when(_valid(t0))
            def _():
                _process(0, t0)

            @pl.when(_valid(t0 + 1))
            def _():
                _process(1, t0 + 1)
            return 0
        lax.fori_loop(0, PAIRS, _pair, 0)

        @pl.when(w <= (N_FULL_CHUNKS - 1) % NW)
        def _():
            _wait_write(0, K_ITERS - 1)

        @pl.when(w > (N_FULL_CHUNKS - 1) % NW)
        def _():
            _wait_write(1, K_ITERS - 2)

        @pl.when(w == 13)
        def _():
            pltpu.sync_copy(hist.at[pl.ds(REM_BASE, REM)],
                            deg0.at[pl.ds(0, REM)])
            mx = jnp.zeros((L,), jnp.int32)
            for i in range(REM // L):
                v = jnp.minimum(deg0[pl.ds(i * L, L)], N_NODES - 1)
                deg0[pl.ds(i * L, L)] = v
                mx = jnp.maximum(mx, v)
            rmax = jnp.max(mx)

            @pl.when(rmax < CACHE_N)
            def _():
                pltpu.async_copy(cache.at[deg0.at[pl.ds(0, REM)]],
                                 buf_e0.at[pl.ds(0, REM)], sem_g0)

            @pl.when(rmax >= CACHE_N)
            def _():
                pltpu.async_copy(emb_hbm.at[deg0.at[pl.ds(0, REM)]],
                                 buf_e0.at[pl.ds(0, REM)], sem_g0)
            cp_x = pltpu.async_copy(x_hbm.at[pl.ds(REM_BASE, REM)],
                                    buf_x0.at[pl.ds(0, REM)], sem_x0)
            pltpu.make_async_copy(emb_hbm.at[deg0.at[pl.ds(0, REM)]],
                                  buf_e0.at[pl.ds(0, REM)], sem_g0).wait()
            cp_x.wait()

            def _radd(r, _):
                for i in range(NODE_DIM // L):
                    plsc.addupdate(buf_e0.at[r, pl.ds(i * L, L)],
                                   buf_x0[r, pl.ds(i * L, L)])
                return 0
            lax.fori_loop(0, REM, _radd, 0)
            pltpu.sync_copy(buf_e0.at[pl.ds(0, REM)],
                            out_hbm.at[pl.ds(REM_BASE, REM)])

    with jax.named_scope("ph_b"):
        pl.run_scoped(_phase_b,
                      pltpu.VMEM((ROWB,), jnp.int32),
                      pltpu.VMEM((ROWB,), jnp.int32),
                      pltpu.VMEM((ROWB, NODE_DIM), jnp.float32),
                      pltpu.VMEM((ROWB, NODE_DIM), jnp.float32),
                      pltpu.VMEM((ROWB, NODE_DIM), jnp.float32),
                      pltpu.VMEM((ROWB, NODE_DIM), jnp.float32))


_sc_call = pl.kernel(
    _body,
    out_type=(jax.ShapeDtypeStruct((N_NODES, NODE_DIM), jnp.float32),
              jax.ShapeDtypeStruct((NC * NS * HIST,), jnp.int32)),
    mesh=plsc.VectorSubcoreMesh(core_axis_name="c", subcore_axis_name="s",
                                num_cores=NC, num_subcores=NS),
    scratch_types=[
        pltpu.VMEM_SHARED((HIST,), jnp.int32),
        pltpu.VMEM_SHARED((CACHE_N, NODE_DIM), jnp.float32),
    ] + [pltpu.SemaphoreType.DMA] * 10,
    compiler_params=pltpu.CompilerParams(needs_layout_passes=False),
)


@jax.jit
def kernel(x, edge_index, degree_embedding):
    edges_flat = edge_index.reshape(-1)
    out, _ = _sc_call(x, edges_flat, degree_embedding)
    return out


# speedup vs baseline: 4.3831x; 1.3320x over previous
import jax
import jax.numpy as jnp
from jax import lax
from jax.experimental import pallas as pl
from jax.experimental.pallas import tpu as pltpu
from jax.experimental.pallas import tpu_sc as plsc

N_NODES = 100000
NODE_DIM = 128
N_EDGES = 1600000

NC, NS, L = 2, 16, 16
NW = NC * NS

CH = 2048
EDGES_PER_TILE = N_EDGES // NS
NCH_FULL = EDGES_PER_TILE // CH
TAIL = EDGES_PER_TILE - NCH_FULL * CH
UNROLL = 4

HIST = 100352
HHALF = HIST // 2
MSLICE = HHALF // NS

CACHE_N = 4096
CROWS = CACHE_N // NS

ROWB = 96
NF0 = HHALF // ROWB
NF1 = (N_NODES - HHALF) // ROWB
REM = HHALF - NF0 * ROWB
REM_BASE = NF0 * ROWB
K_ITERS = (NF0 + NS - 1) // NS
PAIRS = (K_ITERS + 2) // 2


def _body(x_hbm, dst_hbm, emb_hbm, out_hbm, phist_hbm,
          hist, cache, sem_s0, sem_s1, sem_d0, sem_d1, sem_g0, sem_g1,
          sem_x0, sem_x1, sem_w0, sem_w1):
    s = lax.axis_index("s")
    c = lax.axis_index("c")
    sem_s = (sem_s0, sem_s1)

    lo = c * HHALF
    nfull = NF0 - (NF0 - NF1) * c

    pltpu.sync_copy(emb_hbm.at[pl.ds(s * CROWS, CROWS)],
                    cache.at[pl.ds(s * CROWS, CROWS)])

    def _phase_a(hist_priv, st0, st1):
        st = (st0, st1)
        ebase = N_EDGES + s * EDGES_PER_TILE

        def _stage(k, b, sem):
            return pltpu.async_copy(dst_hbm.at[pl.ds(ebase + k * CH, CH)],
                                    st[b], sem)

        ulo = jnp.uint32(lo)
        uhh = jnp.uint32(HHALF)

        with jax.named_scope("ph_a_zero"):
            def _z(i, _):
                for u in range(8):
                    hist_priv[pl.ds((i * 8 + u) * L, L)] = (
                        jnp.zeros((L,), jnp.int32))
                return 0
            lax.fori_loop(0, HHALF // L // 8, _z, 0)

        def _scan(cur, nvec_div):
            def _vec(v, _):
                for u in range(UNROLL):
                    iv = cur[pl.ds((v * UNROLL + u) * L, L)]
                    local = iv - lo
                    inr = (plsc.bitcast(iv, jnp.uint32) - ulo) < uhh
                    cnt, last = plsc.scan_count(iv)
                    plsc.addupdate_scatter(hist_priv, [local], cnt,
                                           mask=last & inr)
                return 0
            lax.fori_loop(0, nvec_div, _vec, 0)

        with jax.named_scope("ph_a_build"):
            _stage(0, 0, sem_s0).wait()

            def _chunk_pair(cp, _):
                k0 = 2 * cp
                _stage(k0 + 1, 1, sem_s1)
                _scan(st0, CH // L // UNROLL)
                pltpu.make_async_copy(
                    dst_hbm.at[pl.ds(ebase, CH)], st1, sem_s1).wait()

                @pl.when(k0 + 2 < NCH_FULL)
                def _():
                    _stage(k0 + 2, 0, sem_s0)
                _scan(st1, CH // L // UNROLL)

                @pl.when(k0 + 2 < NCH_FULL)
                def _():
                    pltpu.make_async_copy(
                        dst_hbm.at[pl.ds(ebase, CH)], st0, sem_s0).wait()
                return 0
            lax.fori_loop(0, NCH_FULL // 2, _chunk_pair, 0)

            pltpu.sync_copy(
                dst_hbm.at[pl.ds(ebase + NCH_FULL * CH, TAIL)],
                st0.at[pl.ds(0, TAIL)])
            _scan(st0, TAIL // L // UNROLL)
            rem_v = (TAIL // L) % UNROLL
            for u in range(rem_v):
                iv = st0[pl.ds((TAIL // L - rem_v + u) * L, L)]
                local = iv - lo
                inr = (plsc.bitcast(iv, jnp.uint32) - ulo) < uhh
                cnt, last = plsc.scan_count(iv)
                plsc.addupdate_scatter(hist_priv, [local], cnt,
                                       mask=last & inr)

            pltpu.sync_copy(
                hist_priv,
                phist_hbm.at[pl.ds((c * NS + s) * HHALF, HHALF)])

        plsc.subcore_barrier()

        acc = hist_priv.at[pl.ds(0, MSLICE)]
        mt = (hist_priv.at[pl.ds(MSLICE, MSLICE)],
              hist_priv.at[pl.ds(2 * MSLICE, MSLICE)])
        my_off = s * MSLICE

        with jax.named_scope("ph_a_merge"):
            def _src(t):
                tt = lax.rem(s + t, NS)
                return phist_hbm.at[
                    pl.ds((c * NS + tt) * HHALF + my_off, MSLICE)]

            pltpu.sync_copy(_src(0), acc)
            pltpu.async_copy(_src(1), mt[1], sem_s[1])
            for t in range(1, NS):
                if t + 1 < NS:
                    pltpu.async_copy(_src(t + 1), mt[(t + 1) % 2],
                                     sem_s[(t + 1) % 2])
                pltpu.make_async_copy(_src(t), mt[t % 2], sem_s[t % 2]).wait()

                def _acc(i, _):
                    plsc.addupdate(acc.at[pl.ds(i * L, L)],
                                   mt[t % 2][pl.ds(i * L, L)])
                    return 0
                lax.fori_loop(0, MSLICE // L, _acc, 0)

            pltpu.sync_copy(acc, hist.at[pl.ds(my_off, MSLICE)])

    pl.run_scoped(_phase_a,
                  pltpu.VMEM((HHALF,), jnp.int32),
                  pltpu.VMEM((CH,), jnp.int32),
                  pltpu.VMEM((CH,), jnp.int32))
    plsc.subcore_barrier()

    def _phase_b(deg0, deg1, buf_e0, buf_e1, buf_x0, buf_x1):
        deg = (deg0, deg1)
        buf_e = (buf_e0, buf_e1)
        buf_x = (buf_x0, buf_x1)
        sem_d = (sem_d0, sem_d1)
        sem_g = (sem_g0, sem_g1)
        sem_x = (sem_x0, sem_x1)
        sem_w = (sem_w0, sem_w1)

        def _valid(t):
            return s + NS * t < nfull

        def _lbase(t):
            return (s + NS * t) * ROWB

        def _stage_deg(b, t):
            pltpu.async_copy(hist.at[pl.ds(_lbase(t), ROWB)],
                             deg[b], sem_d[b])

        def _wait_deg(b, t):
            pltpu.make_async_copy(hist.at[pl.ds(_lbase(t), ROWB)],
                                  deg[b], sem_d[b]).wait()

        def _clamp(b, t):
            mx = jnp.zeros((L,), jnp.int32)
            for i in range(ROWB // L):
                v = jnp.minimum(deg[b][pl.ds(i * L, L)], N_NODES - 1)
                deg[b][pl.ds(i * L, L)] = v
                mx = jnp.maximum(mx, v)
            return jnp.max(mx)

        def _start_gx(b, t, dmax):
            gbase = lo + _lbase(t)

            @pl.when(dmax < CACHE_N)
            def _():
                pltpu.async_copy(cache.at[deg[b]], buf_e[b], sem_g[b])

            @pl.when(dmax >= CACHE_N)
            def _():
                pltpu.async_copy(emb_hbm.at[deg[b]], buf_e[b], sem_g[b])
            pltpu.async_copy(x_hbm.at[pl.ds(gbase, ROWB)], buf_x[b],
                             sem_x[b])

        def _wait_gx(b, t):
            gbase = lo + _lbase(t)
            pltpu.make_async_copy(emb_hbm.at[deg[b]], buf_e[b],
                                  sem_g[b]).wait()
            pltpu.make_async_copy(x_hbm.at[pl.ds(gbase, ROWB)], buf_x[b],
                                  sem_x[b]).wait()

        def _add(b):
            def _rows(r, _):
                for u in range(2):
                    for i in range(NODE_DIM // L):
                        plsc.addupdate(
                            buf_e[b].at[2 * r + u, pl.ds(i * L, L)],
                            buf_x[b][2 * r + u, pl.ds(i * L, L)])
                return 0
            lax.fori_loop(0, ROWB // 2, _rows, 0)

        def _start_write(b, t):
            gbase = lo + _lbase(t)
            pltpu.async_copy(buf_e[b], out_hbm.at[pl.ds(gbase, ROWB)],
                             sem_w[b])

        def _wait_write(b, t):
            gbase = lo + _lbase(t)
            pltpu.make_async_copy(buf_e[b], out_hbm.at[pl.ds(gbase, ROWB)],
                                  sem_w[b]).wait()

        def _process(b, t):
            _wait_gx(b, t)

            @pl.when(_valid(t + 2))
            def _():
                _stage_deg(b, t + 2)

            @pl.when(t >= 1)
            def _():
                _wait_write(1 - b, t - 1)

            @pl.when(_valid(t + 1))
            def _():
                _wait_deg(1 - b, t + 1)
                dmax = _clamp(1 - b, t + 1)
                _start_gx(1 - b, t + 1, dmax)

            _add(b)
            _start_write(b, t)

        _stage_deg(0, 0)

        @pl.when(_valid(1))
        def _():
            _stage_deg(1, 1)
        _wait_deg(0, 0)
        dmax0 = _clamp(0, 0)
        _start_gx(0, 0, dmax0)

        def _pair(tp, _):
            t0 = 2 * tp

            @pl.when(_valid(t0))
            def _():
                _process(0, t0)

            @pl.when(_valid(t0 + 1))
            def _():
                _process(1, t0 + 1)
            return 0
        lax.fori_loop(0, PAIRS, _pair, 0)

        has_full = s < nfull - (K_ITERS - 1) * NS

        @pl.when(has_full)
        def _():
            _wait_write(0, K_ITERS - 1)

        @pl.when(jnp.logical_not(has_full))
        def _():
            _wait_write(1, K_ITERS - 2)

        @pl.when((s == 13) & (c == 0))
        def _():
            pltpu.sync_copy(hist.at[pl.ds(REM_BASE, REM)],
                            deg0.at[pl.ds(0, REM)])
            mx = jnp.zeros((L,), jnp.int32)
            for i in range(REM // L):
                v = jnp.minimum(deg0[pl.ds(i * L, L)], N_NODES - 1)
                deg0[pl.ds(i * L, L)] = v
                mx = jnp.maximum(mx, v)
            rmax = jnp.max(mx)

            @pl.when(rmax < CACHE_N)
            def _():
                pltpu.async_copy(cache.at[deg0.at[pl.ds(0, REM)]],
                                 buf_e0.at[pl.ds(0, REM)], sem_g0)

            @pl.when(rmax >= CACHE_N)
            def _():
                pltpu.async_copy(emb_hbm.at[deg0.at[pl.ds(0, REM)]],
                                 buf_e0.at[pl.ds(0, REM)], sem_g0)
            cp_x = pltpu.async_copy(x_hbm.at[pl.ds(REM_BASE, REM)],
                                    buf_x0.at[pl.ds(0, REM)], sem_x0)
            pltpu.make_async_copy(emb_hbm.at[deg0.at[pl.ds(0, REM)]],
                                  buf_e0.at[pl.ds(0, REM)], sem_g0).wait()
            cp_x.wait()

            def _radd(r, _):
                for i in range(NODE_DIM // L):
                    plsc.addupdate(buf_e0.at[r, pl.ds(i * L, L)],
                                   buf_x0[r, pl.ds(i * L, L)])
                return 0
            lax.fori_loop(0, REM, _radd, 0)
            pltpu.sync_copy(buf_e0.at[pl.ds(0, REM)],
                            out_hbm.at[pl.ds(REM_BASE, REM)])

    with jax.named_scope("ph_b"):
        pl.run_scoped(_phase_b,
                      pltpu.VMEM((ROWB,), jnp.int32),
                      pltpu.VMEM((ROWB,), jnp.int32),
                      pltpu.VMEM((ROWB, NODE_DIM), jnp.float32),
                      pltpu.VMEM((ROWB, NODE_DIM), jnp.float32),
                      pltpu.VMEM((ROWB, NODE_DIM), jnp.float32),
                      pltpu.VMEM((ROWB, NODE_DIM), jnp.float32))


_sc_call = pl.kernel(
    _body,
    out_type=(jax.ShapeDtypeStruct((N_NODES, NODE_DIM), jnp.float32),
              jax.ShapeDtypeStruct((NC * NS * HHALF,), jnp.int32)),
    mesh=plsc.VectorSubcoreMesh(core_axis_name="c", subcore_axis_name="s",
                                num_cores=NC, num_subcores=NS),
    scratch_types=[
        pltpu.VMEM_SHARED((HHALF,), jnp.int32),
        pltpu.VMEM_SHARED((CACHE_N, NODE_DIM), jnp.float32),
    ] + [pltpu.SemaphoreType.DMA] * 10,
    compiler_params=pltpu.CompilerParams(needs_layout_passes=False),
)


@jax.jit
def kernel(x, edge_index, degree_embedding):
    edges_flat = edge_index.reshape(-1)
    out, _ = _sc_call(x, edges_flat, degree_embedding)
    return out
